# baseline (speedup 1.0000x reference)
import functools

import jax
import jax.numpy as jnp
from jax import lax
from jax.experimental import pallas as pl
from jax.experimental.pallas import tpu as pltpu
from jax.experimental.pallas import tpu_sc as plsc

BATCH = 16384
N_DIM = 64
L = 16
NC, NS = 2, 16
NW = NC * NS
CHUNK = 128
ROWS_PER_W = BATCH // NW // CHUNK
GROUPS = CHUNK // L


def _body(u_hbm, i_hbm, a_hbm, uv_hbm, ub_hbm, iv_hbm, ib_hbm,
          c1_hbm, c0_hbm, out_hbm,
          u_v, i_v, a_v, c1_v, c0_v,
          ur0, ur1, ur2, ur3, ir0, ir1, ir2, ir3, ubias, ibias, out_v,
          sems):
    urows = [ur0, ur1, ur2, ur3]
    irows = [ir0, ir1, ir2, ir3]
    wid = lax.axis_index("s") * NC + lax.axis_index("c")
    base = wid * ROWS_PER_W

    pltpu.sync_copy(u_hbm.at[pl.ds(base, ROWS_PER_W)], u_v)
    pltpu.sync_copy(i_hbm.at[pl.ds(base, ROWS_PER_W)], i_v)
    pltpu.sync_copy(a_hbm.at[pl.ds(base, ROWS_PER_W)], a_v)
    pltpu.sync_copy(c1_hbm, c1_v)
    pltpu.sync_copy(c0_hbm, c0_v)

    descs = []
    for j in range(ROWS_PER_W):
        descs.append((
            pltpu.async_copy(uv_hbm.at[u_v.at[j]], urows[j], sems.at[j]),
            pltpu.async_copy(iv_hbm.at[i_v.at[j]], irows[j], sems.at[j]),
            pltpu.async_copy(ub_hbm.at[u_v.at[j]], ubias.at[j], sems.at[j]),
            pltpu.async_copy(ib_hbm.at[i_v.at[j]], ibias.at[j], sems.at[j]),
        ))

    c1v = c1_v[...]
    c0v = c0_v[...]
    lane = jnp.arange(L, dtype=jnp.int32)

    for j in range(ROWS_PER_W):
        for d in descs[j]:
            d.wait()

        def group(g, carry, j=j):
            sl = pl.ds(g * L, L)
            rows = lane + g * L
            accs = [a_v[j, sl] * c1v + c0v,
                    ubias[j, sl] + ibias[j, sl],
                    jnp.zeros((L,), jnp.float32),
                    jnp.zeros((L,), jnp.float32)]
            cols = [jnp.full((L,), k, jnp.int32) for k in range(4)]
            for q in range(N_DIM // 4):
                for k in range(4):
                    cu = plsc.load_gather(urows[j], [rows, cols[k]])
                    ci = plsc.load_gather(irows[j], [rows, cols[k]])
                    accs[k] = accs[k] + cu * ci
                    cols[k] = cols[k] + 4
            out_v[j, sl] = (accs[0] + accs[1]) + (accs[2] + accs[3])
            return carry

        lax.fori_loop(0, GROUPS, group, 0)

    pltpu.sync_copy(out_v, out_hbm.at[pl.ds(base, ROWS_PER_W)])


@jax.jit
def _mfpoly2_sc(u2, i2, a2, user_vect, ub_flat, item_vect, ib_flat, c1, c0):
    mesh = plsc.VectorSubcoreMesh(core_axis_name="c", subcore_axis_name="s")
    f = functools.partial(
        pl.kernel,
        mesh=mesh,
        compiler_params=pltpu.CompilerParams(
            needs_layout_passes=False, use_tc_tiling_on_sc=False),
        out_type=jax.ShapeDtypeStruct((BATCH // CHUNK, CHUNK), jnp.float32),
        scratch_types=[
            pltpu.VMEM((ROWS_PER_W, CHUNK), jnp.int32),
            pltpu.VMEM((ROWS_PER_W, CHUNK), jnp.int32),
            pltpu.VMEM((ROWS_PER_W, CHUNK), jnp.float32),
            pltpu.VMEM((L,), jnp.float32),
            pltpu.VMEM((L,), jnp.float32),
            pltpu.VMEM((CHUNK, N_DIM), jnp.float32),
            pltpu.VMEM((CHUNK, N_DIM), jnp.float32),
            pltpu.VMEM((CHUNK, N_DIM), jnp.float32),
            pltpu.VMEM((CHUNK, N_DIM), jnp.float32),
            pltpu.VMEM((CHUNK, N_DIM), jnp.float32),
            pltpu.VMEM((CHUNK, N_DIM), jnp.float32),
            pltpu.VMEM((CHUNK, N_DIM), jnp.float32),
            pltpu.VMEM((CHUNK, N_DIM), jnp.float32),
            pltpu.VMEM((ROWS_PER_W, CHUNK), jnp.float32),
            pltpu.VMEM((ROWS_PER_W, CHUNK), jnp.float32),
            pltpu.VMEM((ROWS_PER_W, CHUNK), jnp.float32),
            pltpu.SemaphoreType.DMA((ROWS_PER_W,)),
        ],
    )(_body)
    return f(u2, i2, a2, user_vect, ub_flat, item_vect, ib_flat, c1, c0)


def kernel(u, i, a, user_vect, user_bias, item_vect, item_bias, glob_bias,
           age1_w, age1_b, age2_w, age2_b):
    n = u.shape[0]
    u2 = u.astype(jnp.int32).reshape(n // CHUNK, CHUNK)
    i2 = i.astype(jnp.int32).reshape(n // CHUNK, CHUNK)
    a2 = a.reshape(n // CHUNK, CHUNK)
    c1 = age1_w[0, 0] * age2_w[0, 0]
    c0 = glob_bias[0, 0] + age1_b[0] * age2_w[0, 0] + age2_b[0]
    c1v = jnp.full((L,), c1, jnp.float32)
    c0v = jnp.full((L,), c0, jnp.float32)
    out2 = _mfpoly2_sc(u2, i2, a2, user_vect, user_bias.reshape(-1),
                       item_vect, item_bias.reshape(-1), c1v, c0v)
    return out2.reshape(n)
